# Initial kernel scaffold; baseline (speedup 1.0000x reference)
import jax
import jax.numpy as jnp
from jax.experimental import pallas as pl


def kernel(x, table):
    raise NotImplementedError("write your pallas kernel here")


# speedup vs baseline: 3.3431x; 3.3431x over previous
import functools

import jax
import jax.numpy as jnp
from jax import lax
from jax.experimental import pallas as pl
from jax.experimental.pallas import tpu as pltpu
from jax.experimental.pallas import tpu_sc as plsc


def _embed_body(nchunk, chunk, b_per_w, x_hbm, table_hbm, out_hbm,
                idx_v, rows0, rows1, sem0, sem1):
    wid = lax.axis_index("s") * 2 + lax.axis_index("c")
    base = wid * b_per_w
    pltpu.sync_copy(x_hbm.at[pl.ds(base, b_per_w)], idx_v)

    rows = (rows0, rows1)
    sems = (sem0, sem1)

    def gather(g, buf):
        return pltpu.async_copy(
            table_hbm.at[idx_v.at[pl.ds(g * chunk, chunk)]],
            rows[buf], sems[buf])

    gather(0, 0)

    def body(g, _):
        slot = lax.rem(g, 2)
        @pl.when(g + 1 < nchunk)
        def _():
            nslot = lax.rem(g + 1, 2)

            @pl.when(nslot == 0)
            def _():
                gather(g + 1, 0)

            @pl.when(nslot == 1)
            def _():
                gather(g + 1, 1)

        def drain(buf):
            pltpu.make_async_copy(
                table_hbm.at[idx_v.at[pl.ds(g * chunk, chunk)]],
                rows[buf], sems[buf]).wait()
            pltpu.sync_copy(rows[buf],
                            out_hbm.at[pl.ds(base + g * chunk, chunk)])

        @pl.when(slot == 0)
        def _():
            drain(0)

        @pl.when(slot == 1)
        def _():
            drain(1)

        return 0

    lax.fori_loop(0, nchunk, body, 0)


@functools.partial(jax.jit, static_argnames=("b_total", "d", "chunk"))
def _embed(x_flat, table, b_total, d, chunk):
    info = plsc.get_sparse_core_info()
    nw = info.num_cores * info.num_subcores
    b_per_w = b_total // nw
    nchunk = b_per_w // chunk
    mesh = plsc.VectorSubcoreMesh(core_axis_name="c", subcore_axis_name="s")
    kfn = pl.kernel(
        functools.partial(_embed_body, nchunk, chunk, b_per_w),
        mesh=mesh,
        out_type=jax.ShapeDtypeStruct((b_total, d), jnp.float32),
        scratch_types=[
            pltpu.VMEM((b_per_w,), jnp.int32),
            pltpu.VMEM((chunk, d), jnp.float32),
            pltpu.VMEM((chunk, d), jnp.float32),
            pltpu.SemaphoreType.DMA,
            pltpu.SemaphoreType.DMA,
        ],
    )
    return kfn(x_flat, table)


def kernel(x, table):
    b_total = x.shape[0] * x.shape[1]
    d = table.shape[1]
    x_flat = jnp.reshape(x, (b_total,)).astype(jnp.int32)
    out = _embed(x_flat, table, b_total, d, 400)
    return jnp.reshape(out, (*x.shape, d))
